# Initial kernel scaffold; baseline (speedup 1.0000x reference)
import jax
import jax.numpy as jnp
from jax.experimental import pallas as pl


def kernel(node_attr, node_mask, node_weight, root, bias):
    raise NotImplementedError("write your pallas kernel here")


# speedup vs baseline: 1.4110x; 1.4110x over previous
import functools

import jax
import jax.numpy as jnp
from jax.experimental import pallas as pl
from jax.experimental.pallas import tpu as pltpu

_N = 4096
_TILE = 512


def _fused_body(attr_ref, mask_ref, w_ref, root_ref, bias_ref, out_ref, p_ref):
    i = pl.program_id(0)

    @pl.when(i == 0)
    def _():
        p_ref[...] = jnp.dot(attr_ref[...], w_ref[...],
                             preferred_element_type=jnp.float32)

    attr_tile = attr_ref[pl.ds(i * _TILE, _TILE), :]
    out_ref[...] = (
        jnp.dot(mask_ref[...], p_ref[...], preferred_element_type=jnp.float32)
        + jnp.dot(attr_tile, root_ref[...], preferred_element_type=jnp.float32)
        + bias_ref[...]
    )


@jax.jit
def kernel(node_attr, node_mask, node_weight, root, bias):
    n, d_in = node_attr.shape
    d_out = node_weight.shape[1]
    bias2d = bias.reshape(1, d_out)

    grid = (n // _TILE,)
    return pl.pallas_call(
        _fused_body,
        grid=grid,
        in_specs=[
            pl.BlockSpec((n, d_in), lambda i: (0, 0)),
            pl.BlockSpec((_TILE, n), lambda i: (i, 0)),
            pl.BlockSpec((d_in, d_out), lambda i: (0, 0)),
            pl.BlockSpec((d_in, d_out), lambda i: (0, 0)),
            pl.BlockSpec((1, d_out), lambda i: (0, 0)),
        ],
        out_specs=pl.BlockSpec((_TILE, d_out), lambda i: (i, 0)),
        out_shape=jax.ShapeDtypeStruct((n, d_out), jnp.float32),
        scratch_shapes=[pltpu.VMEM((n, d_out), jnp.float32)],
    )(node_attr, node_mask, node_weight, root, bias2d)
